# Initial kernel scaffold; baseline (speedup 1.0000x reference)
import jax
import jax.numpy as jnp
from jax.experimental import pallas as pl


def kernel(sent_embs, lengths):
    raise NotImplementedError("write your pallas kernel here")


# speedup vs baseline: 2.5282x; 2.5282x over previous
import functools

import jax
import jax.numpy as jnp
from jax import lax
from jax.experimental import pallas as pl
from jax.experimental.pallas import tpu as pltpu
from jax.experimental.pallas import tpu_sc as plsc

B = 16
D = 1024
TOTAL = 34816
NC = 2
NS = 16
LANES = 16
HALF = D // NC
ROWS_PER_TILE = TOTAL // NS
CHUNK = 64
NCHUNK = ROWS_PER_TILE // CHUNK
KCOL = HALF // LANES

_mesh = plsc.VectorSubcoreMesh(
    core_axis_name="c", subcore_axis_name="s", num_cores=NC, num_subcores=NS
)


def _body(x_hbm, off_hbm, inv_hbm, out_hbm,
          buf, acc, off_v, inv_v, orow, shacc, sem0, sem1):
    cid = lax.axis_index("c")
    sid = lax.axis_index("s")
    col0 = cid * HALF
    row0 = sid * ROWS_PER_TILE

    pltpu.sync_copy(off_hbm, off_v)
    pltpu.sync_copy(inv_hbm, inv_v)

    zero = jnp.zeros((LANES,), jnp.float32)

    def _zrow(s, c):
        for k in range(KCOL):
            acc[s, pl.ds(k * LANES, LANES)] = zero
        return c

    lax.fori_loop(0, B, _zrow, 0)

    sems = (sem0, sem1)

    def _chunk_copy(j, slot, sem):
        r0 = row0 + j * CHUNK
        return pltpu.make_async_copy(
            x_hbm.at[pl.ds(r0, CHUNK), pl.ds(col0, HALF)], buf.at[slot], sem)

    _chunk_copy(0, 0, sem0).start()
    _chunk_copy(1, 1, sem1).start()

    offs = off_v[...]
    neg1 = jnp.full((LANES,), -1, jnp.int32)
    lane = lax.iota(jnp.int32, LANES)

    def _process(j, slot):
        _chunk_copy(j, slot, sems[slot]).wait()
        r0 = row0 + j * CHUNK
        r0v = jnp.full((LANES,), r0, jnp.int32)
        segv = plsc.all_reduce_population_count(offs <= r0v) + neg1

        def _row(r, c):
            for k in range(KCOL):
                v = buf[slot, r, pl.ds(k * LANES, LANES)]
                plsc.addupdate_scatter(
                    acc, [segv, lane + jnp.full((LANES,), k * LANES, jnp.int32)], v)
            return c

        lax.fori_loop(0, CHUNK, _row, 0)

        @pl.when(j + NC < NCHUNK)
        def _():
            _chunk_copy(j + 2, slot, sems[slot]).start()

    def _pair(t, c):
        _process(2 * t, 0)
        _process(2 * t + 1, 1)
        return c

    lax.fori_loop(0, NCHUNK // 2, _pair, 0)

    pltpu.sync_copy(acc, shacc.at[sid])
    plsc.subcore_barrier()

    for t in range(NS):
        pltpu.async_copy(shacc.at[t, sid], buf.at[0, t, pl.ds(0, HALF)], sem0)
    for t in range(NS):
        pltpu.make_async_copy(
            shacc.at[t, sid], buf.at[0, t, pl.ds(0, HALF)], sem0).wait()

    sidv = jnp.full((LANES,), sid, jnp.int32)
    inv_s = plsc.load_gather(inv_v, [sidv])
    for k in range(KCOL):
        s = buf[0, 0, pl.ds(k * LANES, LANES)]
        for t in range(1, NS):
            s = s + buf[0, t, pl.ds(k * LANES, LANES)]
        orow[pl.ds(k * LANES, LANES)] = s * inv_s
    pltpu.sync_copy(orow, out_hbm.at[sid, pl.ds(col0, HALF)])


_sc_kernel = functools.partial(
    pl.kernel,
    out_type=jax.ShapeDtypeStruct((B, D), jnp.float32),
    mesh=_mesh,
    compiler_params=pltpu.CompilerParams(needs_layout_passes=False),
    scratch_types=[
        pltpu.VMEM((2, CHUNK, HALF), jnp.float32),
        pltpu.VMEM((B, HALF), jnp.float32),
        pltpu.VMEM((LANES,), jnp.int32),
        pltpu.VMEM((LANES,), jnp.float32),
        pltpu.VMEM((HALF,), jnp.float32),
        pltpu.VMEM_SHARED((NS, B, HALF), jnp.float32),
        pltpu.SemaphoreType.DMA,
        pltpu.SemaphoreType.DMA,
    ],
)(_body)


@jax.jit
def kernel(sent_embs, lengths):
    len_i = lengths.astype(jnp.int32)
    off = jnp.concatenate(
        [jnp.zeros((1,), jnp.int32), jnp.cumsum(len_i)[:-1]])
    inv = 1.0 / lengths.astype(jnp.float32)
    return _sc_kernel(sent_embs, off, inv)


# speedup vs baseline: 5.7678x; 2.2814x over previous
import functools

import jax
import jax.numpy as jnp
from jax import lax
from jax.experimental import pallas as pl
from jax.experimental.pallas import tpu as pltpu
from jax.experimental.pallas import tpu_sc as plsc

B = 16
D = 1024
TOTAL = 34816
NC = 2
NS = 16
LANES = 16
HALF = D // NC
ROWS_PER_TILE = TOTAL // NS
CHUNK = 64
NCHUNK = ROWS_PER_TILE // CHUNK
KCOL = HALF // LANES

_mesh = plsc.VectorSubcoreMesh(
    core_axis_name="c", subcore_axis_name="s", num_cores=NC, num_subcores=NS
)


def _body(x_hbm, off_hbm, inv_hbm, out_hbm,
          buf, acc, off_v, inv_v, orow, shacc, sem0, sem1):
    cid = lax.axis_index("c")
    sid = lax.axis_index("s")
    col0 = cid * HALF
    row0 = sid * ROWS_PER_TILE

    pltpu.sync_copy(off_hbm, off_v)
    pltpu.sync_copy(inv_hbm, inv_v)

    zero = jnp.zeros((LANES,), jnp.float32)

    def _zrow(s, c):
        for k in range(KCOL):
            acc[s, pl.ds(k * LANES, LANES)] = zero
        return c

    lax.fori_loop(0, B, _zrow, 0)

    sems = (sem0, sem1)

    def _chunk_copy(j, slot, sem):
        r0 = row0 + j * CHUNK
        return pltpu.make_async_copy(
            x_hbm.at[pl.ds(r0, CHUNK), pl.ds(col0, HALF)], buf.at[slot], sem)

    _chunk_copy(0, 0, sem0).start()
    _chunk_copy(1, 1, sem1).start()

    offs = off_v[...]
    neg1 = jnp.full((LANES,), -1, jnp.int32)
    lane = lax.iota(jnp.int32, LANES)

    RU = 4

    def _process(j, slot):
        _chunk_copy(j, slot, sems[slot]).wait()
        r0 = row0 + j * CHUNK
        r0v = jnp.full((LANES,), r0, jnp.int32)
        segv = plsc.all_reduce_population_count(offs <= r0v) + neg1

        def _rows(i, accs):
            r = i * RU
            new = list(accs)
            for dr in range(RU):
                for k in range(KCOL):
                    new[k] = new[k] + buf[slot, r + dr, pl.ds(k * LANES, LANES)]
            return tuple(new)

        accs = lax.fori_loop(0, CHUNK // RU, _rows, (zero,) * KCOL)
        for k in range(KCOL):
            plsc.addupdate_scatter(
                acc, [segv, lane + jnp.full((LANES,), k * LANES, jnp.int32)],
                accs[k])

        @pl.when(j + NC < NCHUNK)
        def _():
            _chunk_copy(j + 2, slot, sems[slot]).start()

    def _pair(t, c):
        _process(2 * t, 0)
        _process(2 * t + 1, 1)
        return c

    lax.fori_loop(0, NCHUNK // 2, _pair, 0)

    pltpu.sync_copy(acc, shacc.at[sid])
    plsc.subcore_barrier()

    for t in range(NS):
        pltpu.async_copy(shacc.at[t, sid], buf.at[0, t, pl.ds(0, HALF)], sem0)
    for t in range(NS):
        pltpu.make_async_copy(
            shacc.at[t, sid], buf.at[0, t, pl.ds(0, HALF)], sem0).wait()

    sidv = jnp.full((LANES,), sid, jnp.int32)
    inv_s = plsc.load_gather(inv_v, [sidv])
    for k in range(KCOL):
        s = buf[0, 0, pl.ds(k * LANES, LANES)]
        for t in range(1, NS):
            s = s + buf[0, t, pl.ds(k * LANES, LANES)]
        orow[pl.ds(k * LANES, LANES)] = s * inv_s
    pltpu.sync_copy(orow, out_hbm.at[sid, pl.ds(col0, HALF)])


_sc_kernel = functools.partial(
    pl.kernel,
    out_type=jax.ShapeDtypeStruct((B, D), jnp.float32),
    mesh=_mesh,
    compiler_params=pltpu.CompilerParams(needs_layout_passes=False),
    scratch_types=[
        pltpu.VMEM((2, CHUNK, HALF), jnp.float32),
        pltpu.VMEM((B, HALF), jnp.float32),
        pltpu.VMEM((LANES,), jnp.int32),
        pltpu.VMEM((LANES,), jnp.float32),
        pltpu.VMEM((HALF,), jnp.float32),
        pltpu.VMEM_SHARED((NS, B, HALF), jnp.float32),
        pltpu.SemaphoreType.DMA,
        pltpu.SemaphoreType.DMA,
    ],
)(_body)


@jax.jit
def kernel(sent_embs, lengths):
    len_i = lengths.astype(jnp.int32)
    off = jnp.concatenate(
        [jnp.zeros((1,), jnp.int32), jnp.cumsum(len_i)[:-1]])
    inv = 1.0 / lengths.astype(jnp.float32)
    return _sc_kernel(sent_embs, off, inv)
